# Initial kernel scaffold; baseline (speedup 1.0000x reference)
import jax
import jax.numpy as jnp
from jax.experimental import pallas as pl


def kernel(x, batch, att_w):
    raise NotImplementedError("write your pallas kernel here")


# speedup vs baseline: 14.5065x; 14.5065x over previous
import jax
import jax.numpy as jnp
from jax.experimental import pallas as pl
from jax.experimental.pallas import tpu as pltpu

N_TOK = 32768
D = 128
S = 16
BLK = 2048
NB = N_TOK // BLK
NEG = -1e30


def _body(x_ref, b_ref, w_ref, o_ref, acc_ref, m_ref, den_ref, cnt_ref):
    i = pl.program_id(0)

    @pl.when(i == 0)
    def _init():
        acc_ref[...] = jnp.zeros((S, D), jnp.float32)
        m_ref[...] = jnp.full((S, D), NEG, jnp.float32)
        den_ref[...] = jnp.zeros((S, D), jnp.float32)
        cnt_ref[...] = jnp.zeros((S, D), jnp.float32)

    xb = x_ref[...]
    bb = b_ref[0]
    w = w_ref[...]

    seg = jax.lax.broadcasted_iota(jnp.int32, (S, BLK), 0)
    onehot = seg == bb

    s = jax.lax.dot_general(w, xb, (((1,), (1,)), ((), ())),
                            preferred_element_type=jnp.float32)
    s_b = jnp.broadcast_to(s, (S, BLK))

    bmax = jnp.max(jnp.where(onehot, s_b, NEG), axis=1, keepdims=True)
    m_old = jnp.max(m_ref[...], axis=1, keepdims=True)
    m_new = jnp.maximum(m_old, bmax)

    m_tok = jnp.max(jnp.where(onehot, jnp.broadcast_to(m_new, (S, BLK)), NEG),
                    axis=0, keepdims=True)
    p = jnp.exp(s - m_tok)
    ponehot = jnp.where(onehot, jnp.broadcast_to(p, (S, BLK)), 0.0)

    bwsum = jax.lax.dot_general(ponehot, xb, (((1,), (0,)), ((), ())),
                                preferred_element_type=jnp.float32)
    bden = jnp.sum(ponehot, axis=1, keepdims=True)
    bcnt = jnp.sum(onehot.astype(jnp.float32), axis=1, keepdims=True)

    scale = jnp.exp(m_old - m_new)
    acc_ref[...] = acc_ref[...] * scale + bwsum
    den_ref[...] = den_ref[...] * scale + bden
    cnt_ref[...] = cnt_ref[...] + bcnt
    m_ref[...] = jnp.broadcast_to(m_new, (S, D))

    @pl.when(i == NB - 1)
    def _fin():
        o_ref[...] = acc_ref[...] / (den_ref[...] * cnt_ref[...])


def kernel(x, batch, att_w):
    b3 = batch.reshape(NB, 1, BLK)
    g = pl.pallas_call(
        _body,
        grid=(NB,),
        in_specs=[
            pl.BlockSpec((BLK, D), lambda i: (i, 0)),
            pl.BlockSpec((1, 1, BLK), lambda i: (i, 0, 0)),
            pl.BlockSpec((1, D), lambda i: (0, 0)),
        ],
        out_specs=pl.BlockSpec((S, D), lambda i: (0, 0)),
        out_shape=jax.ShapeDtypeStruct((S, D), jnp.float32),
        scratch_shapes=[
            pltpu.VMEM((S, D), jnp.float32),
            pltpu.VMEM((S, D), jnp.float32),
            pltpu.VMEM((S, D), jnp.float32),
            pltpu.VMEM((S, D), jnp.float32),
        ],
    )(x, b3, att_w)
    return (g, att_w)
